# baseline (speedup 1.0000x reference)
import functools

import jax
import jax.numpy as jnp
from jax import lax
from jax.experimental import pallas as pl
from jax.experimental.pallas import tpu as pltpu
from jax.experimental.pallas import tpu_sc as plsc

N = 10000
E = 320000
D_IN = 128
H1 = 64
H2 = 64
AW = 80
NC = 2
NS = 16
L = 16
NW = NC * NS
EW = E // NW
CH = 80
NCHUNK = EW // CH
NPAIR = (NCHUNK - 1) // 2
NPT_A = 624
NPT_B = N - (NS - 1) * NPT_A
NZB = 208

_SC_MESH = plsc.VectorSubcoreMesh(core_axis_name="c", subcore_axis_name="s")

_SC_PARAMS = pltpu.CompilerParams(
    needs_layout_passes=False, use_tc_tiling_on_sc=False)


def _node_pre_body(x_ref, wn_ref, bn_ref, av_ref, xa_ref):
  xn = jnp.dot(x_ref[...], wn_ref[...], preferred_element_type=jnp.float32)
  xn = xn + bn_ref[...]
  a = jnp.dot(xn, av_ref[...], preferred_element_type=jnp.float32)
  rb = xn.shape[0]
  xa_ref[:, :H1] = xn
  xa_ref[:, H1:H1 + 1] = jnp.ones((rb, 1), jnp.float32)
  xa_ref[:, H1 + 1:H1 + 2] = a
  xa_ref[:, H1 + 2:] = jnp.zeros((rb, AW - H1 - 2), jnp.float32)


def _node_pre(x, wn, bn2, av2):
  rb = 1000
  return pl.pallas_call(
      _node_pre_body,
      grid=(N // rb,),
      in_specs=[
          pl.BlockSpec((rb, D_IN), lambda i: (i, 0)),
          pl.BlockSpec((D_IN, H1), lambda i: (0, 0)),
          pl.BlockSpec((1, H1), lambda i: (0, 0)),
          pl.BlockSpec((H1, 1), lambda i: (0, 0)),
      ],
      out_specs=pl.BlockSpec((rb, AW), lambda i: (i, 0)),
      out_shape=jax.ShapeDtypeStruct((N, AW), jnp.float32),
  )(x, wn, bn2, av2)


def _gate_pre_body(ea_ref, m1_ref, b1c_ref, m2_ref, cb2_ref,
                   fcw2_ref, fcb2_ref, wa_ref, ba_ref, gate_ref, ae_ref):
  u = ea_ref[...]
  c1 = jnp.dot(u, m1_ref[...], preferred_element_type=jnp.float32)
  c1 = c1 + b1c_ref[...]
  c1 = jnp.where(c1 >= 0, c1, 0.1 * c1)
  c2 = jnp.dot(c1, m2_ref[...], preferred_element_type=jnp.float32)
  c2 = c2 + cb2_ref[0, 0]
  c2 = jnp.where(c2 >= 0, c2, 0.1 * c2)
  eij = jnp.dot(c2, fcw2_ref[...], preferred_element_type=jnp.float32)
  eij = eij + fcb2_ref[...]
  gate_ref[...] = jnp.maximum(eij, 0.0) + jnp.log1p(jnp.exp(-jnp.abs(eij)))
  ae = jnp.dot(u, wa_ref[...], preferred_element_type=jnp.float32)
  ae_ref[...] = ae + ba_ref[...]


def _gate_pre(ea, m1, b1c2, m2, cb22, fcw2, fcb2w, wa2, ba2):
  eb = 2000
  return pl.pallas_call(
      _gate_pre_body,
      grid=(E // eb,),
      in_specs=[
          pl.BlockSpec((eb, 16), lambda i: (i, 0)),
          pl.BlockSpec((16, H1), lambda i: (0, 0)),
          pl.BlockSpec((1, H1), lambda i: (0, 0)),
          pl.BlockSpec((H1, 8), lambda i: (0, 0)),
          pl.BlockSpec((1, 1), lambda i: (0, 0)),
          pl.BlockSpec((8, 2 * H2), lambda i: (0, 0)),
          pl.BlockSpec((1, 2 * H2), lambda i: (0, 0)),
          pl.BlockSpec((16, 1), lambda i: (0, 0)),
          pl.BlockSpec((1, 1), lambda i: (0, 0)),
      ],
      out_specs=[
          pl.BlockSpec((eb, 2 * H2), lambda i: (i, 0)),
          pl.BlockSpec((eb, 1), lambda i: (i, 0)),
      ],
      out_shape=[
          jax.ShapeDtypeStruct((E, 2 * H2), jnp.float32),
          jax.ShapeDtypeStruct((E, 1), jnp.float32),
      ],
  )(ea, m1, b1c2, m2, cb22, fcw2, fcb2w, wa2, ba2)


def _gat_sc_body(xa_hbm, src2_hbm, dst2_hbm, ae2_hbm, out_hbm,
                 srcb_v, dstb_v, aeb_v, coef_v, rows_a, rows_b, msg_a, msg_b,
                 zb_v, acc_sh, lsem, ga, gb, sa, sb):
  cid = lax.axis_index("c")
  sid = lax.axis_index("s")
  wid = cid * NS + sid
  row0 = wid * NCHUNK

  d1 = pltpu.async_copy(src2_hbm.at[pl.ds(row0, NCHUNK)], srcb_v, lsem)
  d2 = pltpu.async_copy(dst2_hbm.at[pl.ds(row0, NCHUNK)], dstb_v, lsem)
  d3 = pltpu.async_copy(ae2_hbm.at[pl.ds(row0, NCHUNK)], aeb_v, lsem)

  @pl.loop(0, NZB)
  def _(i):
    for c in range(AW // L):
      zb_v[i, pl.ds(c * L, L)] = jnp.zeros((L,), jnp.float32)

  for m in range(NPT_A // NZB):
    pltpu.sync_copy(zb_v, acc_sh.at[pl.ds(sid * NPT_A + m * NZB, NZB)])

  @pl.when(sid == NS - 1)
  def _():
    pltpu.sync_copy(zb_v.at[pl.ds(0, NPT_B - NPT_A)],
                    acc_sh.at[pl.ds(N - (NPT_B - NPT_A), NPT_B - NPT_A)])

  d1.wait()
  d2.wait()
  d3.wait()
  plsc.subcore_barrier()

  def compute(ci, rows, msg):
    for c in range(CH // L):
      jv = lax.iota(jnp.int32, L) + (c * L)
      an = plsc.load_gather(rows, [jv, jnp.full((L,), H1 + 1, jnp.int32)])
      t = an + aeb_v[ci, pl.ds(c * L, L)]
      t = jnp.where(t >= 0, t, 0.01 * t)
      coef_v[pl.ds(c * L, L)] = jnp.exp(t)

    @pl.loop(0, CH, unroll=8)
    def _(j):
      s = coef_v[pl.ds(j, L)][0]
      for c in range(AW // L):
        sl = pl.ds(c * L, L)
        msg[j, sl] = rows[j, sl] * s

  pltpu.async_copy(xa_hbm.at[srcb_v.at[0]], rows_a, ga)

  @pl.loop(0, NPAIR)
  def _(k):
    c0 = 2 * k
    pltpu.async_copy(xa_hbm.at[srcb_v.at[c0 + 1]], rows_b, gb)
    pltpu.make_async_copy(xa_hbm.at[srcb_v.at[c0]], rows_a, ga).wait()

    @pl.when(k > 0)
    def _():
      pltpu.make_async_copy(msg_a, acc_sh.at[dstb_v.at[c0 - 2]], sa).wait()

    compute(c0, rows_a, msg_a)
    pltpu.async_copy(msg_a, acc_sh.at[dstb_v.at[c0]], sa, add=True)
    pltpu.async_copy(xa_hbm.at[srcb_v.at[c0 + 2]], rows_a, ga)
    pltpu.make_async_copy(xa_hbm.at[srcb_v.at[c0 + 1]], rows_b, gb).wait()

    @pl.when(k > 0)
    def _():
      pltpu.make_async_copy(msg_b, acc_sh.at[dstb_v.at[c0 - 1]], sb).wait()

    compute(c0 + 1, rows_b, msg_b)
    pltpu.async_copy(msg_b, acc_sh.at[dstb_v.at[c0 + 1]], sb, add=True)

  last = NCHUNK - 1
  pltpu.make_async_copy(xa_hbm.at[srcb_v.at[last]], rows_a, ga).wait()
  pltpu.make_async_copy(msg_a, acc_sh.at[dstb_v.at[last - 2]], sa).wait()
  compute(last, rows_a, msg_a)
  pltpu.async_copy(msg_a, acc_sh.at[dstb_v.at[last]], sa, add=True)
  pltpu.make_async_copy(msg_a, acc_sh.at[dstb_v.at[last]], sa).wait()
  pltpu.make_async_copy(msg_b, acc_sh.at[dstb_v.at[last - 1]], sb).wait()

  plsc.subcore_barrier()

  for m in range(NPT_A // NZB):
    pltpu.sync_copy(acc_sh.at[pl.ds(sid * NPT_A + m * NZB, NZB)], zb_v)
    pltpu.sync_copy(zb_v, out_hbm.at[cid, pl.ds(sid * NPT_A + m * NZB, NZB)])

  @pl.when(sid == NS - 1)
  def _():
    pltpu.sync_copy(acc_sh.at[pl.ds(N - (NPT_B - NPT_A), NPT_B - NPT_A)],
                    zb_v.at[pl.ds(0, NPT_B - NPT_A)])
    pltpu.sync_copy(zb_v.at[pl.ds(0, NPT_B - NPT_A)],
                    out_hbm.at[cid, pl.ds(N - (NPT_B - NPT_A), NPT_B - NPT_A)])


def _gat_sc(xa, src2, dst2, ae2):
  k = pl.kernel(
      _gat_sc_body,
      out_type=jax.ShapeDtypeStruct((NC, N, AW), jnp.float32),
      mesh=_SC_MESH,
      scratch_types=[
          pltpu.VMEM((NCHUNK, CH), jnp.int32),
          pltpu.VMEM((NCHUNK, CH), jnp.int32),
          pltpu.VMEM((NCHUNK, CH), jnp.float32),
          pltpu.VMEM((CH + L,), jnp.float32),
          pltpu.VMEM((CH, AW), jnp.float32),
          pltpu.VMEM((CH, AW), jnp.float32),
          pltpu.VMEM((CH, AW), jnp.float32),
          pltpu.VMEM((CH, AW), jnp.float32),
          pltpu.VMEM((NZB, AW), jnp.float32),
          pltpu.VMEM_SHARED((N, AW), jnp.float32),
          pltpu.SemaphoreType.DMA,
          pltpu.SemaphoreType.DMA,
          pltpu.SemaphoreType.DMA,
          pltpu.SemaphoreType.DMA,
          pltpu.SemaphoreType.DMA,
      ],
      compiler_params=_SC_PARAMS,
  )
  return k(xa, src2, dst2, ae2)


def _mid_body(acc_ref, w1_ref, b1_ref, w2_ref, b2_ref, w2p_ref, hw1_ref):
  s = acc_ref[0] + acc_ref[1]
  h = s[:, :H1] / (s[:, H1:H1 + 1] + 1e-16)
  h = jnp.maximum(h, 0.0)
  w2p_ref[...] = jnp.dot(h, w2_ref[...],
                         preferred_element_type=jnp.float32) + b2_ref[...]
  hw1_ref[...] = jnp.dot(h, w1_ref[...],
                         preferred_element_type=jnp.float32) + b1_ref[...]


def _mid(acc, w1, b12, w2, b22):
  rb = 1000
  return pl.pallas_call(
      _mid_body,
      grid=(N // rb,),
      in_specs=[
          pl.BlockSpec((NC, rb, AW), lambda i: (0, i, 0)),
          pl.BlockSpec((H1, H2), lambda i: (0, 0)),
          pl.BlockSpec((1, H2), lambda i: (0, 0)),
          pl.BlockSpec((H1, H2), lambda i: (0, 0)),
          pl.BlockSpec((1, H2), lambda i: (0, 0)),
      ],
      out_specs=[
          pl.BlockSpec((rb, H2), lambda i: (i, 0)),
          pl.BlockSpec((rb, H2), lambda i: (i, 0)),
      ],
      out_shape=[
          jax.ShapeDtypeStruct((N, H2), jnp.float32),
          jax.ShapeDtypeStruct((N, H2), jnp.float32),
      ],
  )(acc, w1, b12, w2, b22)


def _aggr_sc_body(w2p_hbm, src2_hbm, dst2_hbm, gate_hbm, out_hbm,
                  srcb_v, dstb_v, rows_a, rows_b, gate_a, gate_b,
                  msg_a, msg_b, zb_v, acc_sh, lsem, ga, gb, ha, hb, sa, sb):
  cid = lax.axis_index("c")
  sid = lax.axis_index("s")
  wid = cid * NS + sid
  row0 = wid * NCHUNK
  ebase = wid * EW

  d1 = pltpu.async_copy(src2_hbm.at[pl.ds(row0, NCHUNK)], srcb_v, lsem)
  d2 = pltpu.async_copy(dst2_hbm.at[pl.ds(row0, NCHUNK)], dstb_v, lsem)

  @pl.loop(0, NZB)
  def _(i):
    for c in range(H2 // L):
      zb_v[i, pl.ds(c * L, L)] = jnp.zeros((L,), jnp.float32)

  for m in range(NPT_A // NZB):
    pltpu.sync_copy(zb_v, acc_sh.at[pl.ds(sid * NPT_A + m * NZB, NZB)])

  @pl.when(sid == NS - 1)
  def _():
    pltpu.sync_copy(zb_v.at[pl.ds(0, NPT_B - NPT_A)],
                    acc_sh.at[pl.ds(N - (NPT_B - NPT_A), NPT_B - NPT_A)])

  d1.wait()
  d2.wait()
  plsc.subcore_barrier()

  def compute(rows, gate, msg):
    @pl.loop(0, CH, unroll=8)
    def _(j):
      for c in range(H2 // L):
        sl = pl.ds(c * L, L)
        msg[j, sl] = rows[j, sl] * gate[j, sl]

  pltpu.async_copy(w2p_hbm.at[srcb_v.at[0]], rows_a, ga)
  pltpu.async_copy(gate_hbm.at[pl.ds(ebase, CH)], gate_a, ha)

  @pl.loop(0, NPAIR)
  def _(k):
    c0 = 2 * k
    pltpu.async_copy(w2p_hbm.at[srcb_v.at[c0 + 1]], rows_b, gb)
    pltpu.async_copy(gate_hbm.at[pl.ds(ebase + (c0 + 1) * CH, CH)], gate_b, hb)
    pltpu.make_async_copy(w2p_hbm.at[srcb_v.at[c0]], rows_a, ga).wait()
    pltpu.make_async_copy(gate_hbm.at[pl.ds(ebase, CH)], gate_a, ha).wait()

    @pl.when(k > 0)
    def _():
      pltpu.make_async_copy(msg_a, acc_sh.at[dstb_v.at[c0 - 2]], sa).wait()

    compute(rows_a, gate_a, msg_a)
    pltpu.async_copy(msg_a, acc_sh.at[dstb_v.at[c0]], sa, add=True)
    pltpu.async_copy(w2p_hbm.at[srcb_v.at[c0 + 2]], rows_a, ga)
    pltpu.async_copy(gate_hbm.at[pl.ds(ebase + (c0 + 2) * CH, CH)], gate_a, ha)
    pltpu.make_async_copy(w2p_hbm.at[srcb_v.at[c0 + 1]], rows_b, gb).wait()
    pltpu.make_async_copy(gate_hbm.at[pl.ds(ebase, CH)], gate_b, hb).wait()

    @pl.when(k > 0)
    def _():
      pltpu.make_async_copy(msg_b, acc_sh.at[dstb_v.at[c0 - 1]], sb).wait()

    compute(rows_b, gate_b, msg_b)
    pltpu.async_copy(msg_b, acc_sh.at[dstb_v.at[c0 + 1]], sb, add=True)

  last = NCHUNK - 1
  pltpu.make_async_copy(w2p_hbm.at[srcb_v.at[last]], rows_a, ga).wait()
  pltpu.make_async_copy(gate_hbm.at[pl.ds(ebase, CH)], gate_a, ha).wait()
  pltpu.make_async_copy(msg_a, acc_sh.at[dstb_v.at[last - 2]], sa).wait()
  compute(rows_a, gate_a, msg_a)
  pltpu.async_copy(msg_a, acc_sh.at[dstb_v.at[last]], sa, add=True)
  pltpu.make_async_copy(msg_a, acc_sh.at[dstb_v.at[last]], sa).wait()
  pltpu.make_async_copy(msg_b, acc_sh.at[dstb_v.at[last - 1]], sb).wait()

  plsc.subcore_barrier()

  for m in range(NPT_A // NZB):
    pltpu.sync_copy(acc_sh.at[pl.ds(sid * NPT_A + m * NZB, NZB)], zb_v)
    pltpu.sync_copy(zb_v, out_hbm.at[cid, pl.ds(sid * NPT_A + m * NZB, NZB)])

  @pl.when(sid == NS - 1)
  def _():
    pltpu.sync_copy(acc_sh.at[pl.ds(N - (NPT_B - NPT_A), NPT_B - NPT_A)],
                    zb_v.at[pl.ds(0, NPT_B - NPT_A)])
    pltpu.sync_copy(zb_v.at[pl.ds(0, NPT_B - NPT_A)],
                    out_hbm.at[cid, pl.ds(N - (NPT_B - NPT_A), NPT_B - NPT_A)])


def _aggr_sc(w2p, src2, dst2, gate):
  k = pl.kernel(
      _aggr_sc_body,
      out_type=jax.ShapeDtypeStruct((NC, N, H2), jnp.float32),
      mesh=_SC_MESH,
      scratch_types=[
          pltpu.VMEM((NCHUNK, CH), jnp.int32),
          pltpu.VMEM((NCHUNK, CH), jnp.int32),
          pltpu.VMEM((CH, H2), jnp.float32),
          pltpu.VMEM((CH, H2), jnp.float32),
          pltpu.VMEM((CH, 2 * H2), jnp.float32),
          pltpu.VMEM((CH, 2 * H2), jnp.float32),
          pltpu.VMEM((CH, H2), jnp.float32),
          pltpu.VMEM((CH, H2), jnp.float32),
          pltpu.VMEM((NZB, H2), jnp.float32),
          pltpu.VMEM_SHARED((N, H2), jnp.float32),
          pltpu.SemaphoreType.DMA,
          pltpu.SemaphoreType.DMA,
          pltpu.SemaphoreType.DMA,
          pltpu.SemaphoreType.DMA,
          pltpu.SemaphoreType.DMA,
          pltpu.SemaphoreType.DMA,
          pltpu.SemaphoreType.DMA,
      ],
      compiler_params=_SC_PARAMS,
  )
  return k(w2p, src2, dst2, gate)


def _head_body(hw1_ref, aggr_ref, g_ref, b_ref, out_ref):
  hh = hw1_ref[...] + aggr_ref[0] + aggr_ref[1]
  mean = jnp.mean(hh, axis=-1, keepdims=True)
  d = hh - mean
  var = jnp.mean(d * d, axis=-1, keepdims=True)
  y = d * lax.rsqrt(var + 1e-5) * g_ref[...] + b_ref[...]
  y = jnp.maximum(y, 0.0)
  m = jnp.max(y, axis=-1, keepdims=True)
  ey = jnp.exp(y - m)
  sm = ey / jnp.sum(ey, axis=-1, keepdims=True)
  o = jnp.sqrt(sm + 1e-8)
  nrm = jnp.sqrt(jnp.sum(o * o, axis=-1, keepdims=True))
  out_ref[...] = o / nrm


def _head(hw1, aggr, g2, b2):
  rb = 1000
  return pl.pallas_call(
      _head_body,
      grid=(N // rb,),
      in_specs=[
          pl.BlockSpec((rb, H2), lambda i: (i, 0)),
          pl.BlockSpec((NC, rb, H2), lambda i: (0, i, 0)),
          pl.BlockSpec((1, H2), lambda i: (0, 0)),
          pl.BlockSpec((1, H2), lambda i: (0, 0)),
      ],
      out_specs=pl.BlockSpec((rb, H2), lambda i: (i, 0)),
      out_shape=jax.ShapeDtypeStruct((N, H2), jnp.float32),
  )(hw1, aggr, g2, b2)


def kernel(x, edge_index, edge_attr, Wn, bn, We, be, attn, W1, b1, W2, b2,
           cw1, cb1, cw2, cb2, fcW, fcb, gamma, beta):
  av = attn[0, 0]
  wa = We @ av
  ba = be @ av
  m1 = jnp.zeros((16, H1), jnp.float32)
  m2 = jnp.zeros((H1, 8), jnp.float32)
  for t in range(3):
    s_t = jnp.eye(8, 8, 1 - t, dtype=jnp.float32)
    m1 = m1 + jnp.kron(cw1[:, :, t].T, s_t)
    m2 = m2 + jnp.kron(cw2[0, :, t:t + 1], s_t)
  b1c = jnp.repeat(cb1, 8)

  xa = _node_pre(x, Wn, bn.reshape(1, H1), av.reshape(H1, 1))
  fcw2 = jnp.concatenate([fcW, fcW], axis=1)
  fcb2w = jnp.concatenate([fcb, fcb]).reshape(1, 2 * H2)
  gate, ae = _gate_pre(edge_attr, m1, b1c.reshape(1, H1), m2,
                       cb2.reshape(1, 1), fcw2, fcb2w,
                       wa.reshape(16, 1), ba.reshape(1, 1))
  src2 = edge_index[0].reshape(E // CH, CH)
  dst2 = edge_index[1].reshape(E // CH, CH)
  acc = _gat_sc(xa, src2, dst2, ae.reshape(E // CH, CH))
  w2p, hw1 = _mid(acc, W1, b1.reshape(1, H2), W2, b2.reshape(1, H2))
  aggr = _aggr_sc(w2p, src2, dst2, gate)
  return _head(hw1, aggr, gamma.reshape(1, H2), beta.reshape(1, H2))


# speedup vs baseline: 1.3056x; 1.3056x over previous
import functools

import jax
import jax.numpy as jnp
from jax import lax
from jax.experimental import pallas as pl
from jax.experimental.pallas import tpu as pltpu
from jax.experimental.pallas import tpu_sc as plsc

N = 10000
E = 320000
D_IN = 128
H1 = 64
H2 = 64
AW = 80
NC = 2
NS = 16
L = 16
NW = NC * NS
EW = E // NW
CH = 80
NCHUNK = EW // CH
NPAIR = (NCHUNK - 1) // 2
NPT_A = 624
NPT_B = N - (NS - 1) * NPT_A
NZB = 208

_SC_MESH = plsc.VectorSubcoreMesh(core_axis_name="c", subcore_axis_name="s")

_SC_PARAMS = pltpu.CompilerParams(
    needs_layout_passes=False, use_tc_tiling_on_sc=False)


def _node_pre_body(x_ref, wn_ref, bn_ref, av_ref, xa_ref):
  xn = jnp.dot(x_ref[...], wn_ref[...], preferred_element_type=jnp.float32)
  xn = xn + bn_ref[...]
  a = jnp.dot(xn, av_ref[...], preferred_element_type=jnp.float32)
  rb = xn.shape[0]
  xa_ref[:, :H1] = xn
  xa_ref[:, H1:H1 + 1] = jnp.ones((rb, 1), jnp.float32)
  xa_ref[:, H1 + 1:H1 + 2] = a
  xa_ref[:, H1 + 2:] = jnp.zeros((rb, AW - H1 - 2), jnp.float32)


def _node_pre(x, wn, bn2, av2):
  rb = 1000
  return pl.pallas_call(
      _node_pre_body,
      grid=(N // rb,),
      in_specs=[
          pl.BlockSpec((rb, D_IN), lambda i: (i, 0)),
          pl.BlockSpec((D_IN, H1), lambda i: (0, 0)),
          pl.BlockSpec((1, H1), lambda i: (0, 0)),
          pl.BlockSpec((H1, 1), lambda i: (0, 0)),
      ],
      out_specs=pl.BlockSpec((rb, AW), lambda i: (i, 0)),
      out_shape=jax.ShapeDtypeStruct((N, AW), jnp.float32),
  )(x, wn, bn2, av2)


def _ae_pre_body(ea_ref, wa_ref, ba_ref, ae_ref):
  u = ea_ref[...]
  ae = jnp.dot(u, wa_ref[...], preferred_element_type=jnp.float32)
  ae = ae + ba_ref[...]
  ae_ref[...] = ae.reshape(ae_ref.shape)


def _ae_pre(ea, wa2, ba2):
  eb = 6400
  return pl.pallas_call(
      _ae_pre_body,
      grid=(E // eb,),
      in_specs=[
          pl.BlockSpec((eb, 16), lambda i: (i, 0)),
          pl.BlockSpec((16, 1), lambda i: (0, 0)),
          pl.BlockSpec((1, 1), lambda i: (0, 0)),
      ],
      out_specs=pl.BlockSpec((eb // CH, CH), lambda i: (i, 0)),
      out_shape=jax.ShapeDtypeStruct((E // CH, CH), jnp.float32),
  )(ea, wa2, ba2)


def _gate_pre_body(ea_ref, m1_ref, b1c_ref, m2_ref, cb2_ref,
                   fcw2_ref, fcb2_ref, gate_ref):
  u = ea_ref[...]
  c1 = jnp.dot(u, m1_ref[...], preferred_element_type=jnp.float32)
  c1 = c1 + b1c_ref[...]
  c1 = jnp.where(c1 >= 0, c1, 0.1 * c1)
  c2 = jnp.dot(c1, m2_ref[...], preferred_element_type=jnp.float32)
  c2 = c2 + cb2_ref[0, 0]
  c2 = jnp.where(c2 >= 0, c2, 0.1 * c2)
  eij = jnp.dot(c2, fcw2_ref[...], preferred_element_type=jnp.float32)
  eij = eij + fcb2_ref[...]
  gate_ref[...] = jnp.maximum(eij, 0.0) + jnp.log1p(jnp.exp(-jnp.abs(eij)))


def _gate_pre(ea, m1, b1c2, m2, cb22, fcw2, fcb2w):
  eb = 2000
  return pl.pallas_call(
      _gate_pre_body,
      grid=(E // eb,),
      in_specs=[
          pl.BlockSpec((eb, 16), lambda i: (i, 0)),
          pl.BlockSpec((16, H1), lambda i: (0, 0)),
          pl.BlockSpec((1, H1), lambda i: (0, 0)),
          pl.BlockSpec((H1, 8), lambda i: (0, 0)),
          pl.BlockSpec((1, 1), lambda i: (0, 0)),
          pl.BlockSpec((8, 2 * H2), lambda i: (0, 0)),
          pl.BlockSpec((1, 2 * H2), lambda i: (0, 0)),
      ],
      out_specs=pl.BlockSpec((eb, 2 * H2), lambda i: (i, 0)),
      out_shape=jax.ShapeDtypeStruct((E, 2 * H2), jnp.float32),
  )(ea, m1, b1c2, m2, cb22, fcw2, fcb2w)


def _gat_sc_body(xa_hbm, src2_hbm, dst2_hbm, ae2_hbm, out_hbm,
                 srcb_v, dstb_v, aeb_v, coef_v, rows_a, rows_b, msg_a, msg_b,
                 zb_v, acc_sh, lsem, ga, gb, sa, sb):
  cid = lax.axis_index("c")
  sid = lax.axis_index("s")
  wid = cid * NS + sid
  row0 = wid * NCHUNK

  d1 = pltpu.async_copy(src2_hbm.at[pl.ds(row0, NCHUNK)], srcb_v, lsem)
  d2 = pltpu.async_copy(dst2_hbm.at[pl.ds(row0, NCHUNK)], dstb_v, lsem)
  d3 = pltpu.async_copy(ae2_hbm.at[pl.ds(row0, NCHUNK)], aeb_v, lsem)

  @pl.loop(0, NZB)
  def _(i):
    for c in range(AW // L):
      zb_v[i, pl.ds(c * L, L)] = jnp.zeros((L,), jnp.float32)

  for m in range(NPT_A // NZB):
    pltpu.sync_copy(zb_v, acc_sh.at[pl.ds(sid * NPT_A + m * NZB, NZB)])

  @pl.when(sid == NS - 1)
  def _():
    pltpu.sync_copy(zb_v.at[pl.ds(0, NPT_B - NPT_A)],
                    acc_sh.at[pl.ds(N - (NPT_B - NPT_A), NPT_B - NPT_A)])

  d1.wait()
  d2.wait()
  d3.wait()
  plsc.subcore_barrier()

  def compute(ci, rows, msg):
    for c in range(CH // L):
      jv = lax.iota(jnp.int32, L) + (c * L)
      an = plsc.load_gather(rows, [jv, jnp.full((L,), H1 + 1, jnp.int32)])
      t = an + aeb_v[ci, pl.ds(c * L, L)]
      t = jnp.where(t >= 0, t, 0.01 * t)
      coef_v[pl.ds(c * L, L)] = jnp.exp(t)

    @pl.loop(0, CH, unroll=8)
    def _(j):
      s = coef_v[pl.ds(j, L)][0]
      for c in range(AW // L):
        sl = pl.ds(c * L, L)
        msg[j, sl] = rows[j, sl] * s

  pltpu.async_copy(xa_hbm.at[srcb_v.at[0]], rows_a, ga)

  @pl.loop(0, NPAIR)
  def _(k):
    c0 = 2 * k
    pltpu.async_copy(xa_hbm.at[srcb_v.at[c0 + 1]], rows_b, gb)
    pltpu.make_async_copy(xa_hbm.at[srcb_v.at[c0]], rows_a, ga).wait()

    @pl.when(k > 0)
    def _():
      pltpu.make_async_copy(msg_a, acc_sh.at[dstb_v.at[c0 - 2]], sa).wait()

    compute(c0, rows_a, msg_a)
    pltpu.async_copy(msg_a, acc_sh.at[dstb_v.at[c0]], sa, add=True)
    pltpu.async_copy(xa_hbm.at[srcb_v.at[c0 + 2]], rows_a, ga)
    pltpu.make_async_copy(xa_hbm.at[srcb_v.at[c0 + 1]], rows_b, gb).wait()

    @pl.when(k > 0)
    def _():
      pltpu.make_async_copy(msg_b, acc_sh.at[dstb_v.at[c0 - 1]], sb).wait()

    compute(c0 + 1, rows_b, msg_b)
    pltpu.async_copy(msg_b, acc_sh.at[dstb_v.at[c0 + 1]], sb, add=True)

  last = NCHUNK - 1
  pltpu.make_async_copy(xa_hbm.at[srcb_v.at[last]], rows_a, ga).wait()
  pltpu.make_async_copy(msg_a, acc_sh.at[dstb_v.at[last - 2]], sa).wait()
  compute(last, rows_a, msg_a)
  pltpu.async_copy(msg_a, acc_sh.at[dstb_v.at[last]], sa, add=True)
  pltpu.make_async_copy(msg_a, acc_sh.at[dstb_v.at[last]], sa).wait()
  pltpu.make_async_copy(msg_b, acc_sh.at[dstb_v.at[last - 1]], sb).wait()

  plsc.subcore_barrier()

  for m in range(NPT_A // NZB):
    pltpu.sync_copy(acc_sh.at[pl.ds(sid * NPT_A + m * NZB, NZB)], zb_v)
    pltpu.sync_copy(zb_v, out_hbm.at[cid, pl.ds(sid * NPT_A + m * NZB, NZB)])

  @pl.when(sid == NS - 1)
  def _():
    pltpu.sync_copy(acc_sh.at[pl.ds(N - (NPT_B - NPT_A), NPT_B - NPT_A)],
                    zb_v.at[pl.ds(0, NPT_B - NPT_A)])
    pltpu.sync_copy(zb_v.at[pl.ds(0, NPT_B - NPT_A)],
                    out_hbm.at[cid, pl.ds(N - (NPT_B - NPT_A), NPT_B - NPT_A)])


def _gat_sc(xa, src2, dst2, ae2):
  k = pl.kernel(
      _gat_sc_body,
      out_type=jax.ShapeDtypeStruct((NC, N, AW), jnp.float32),
      mesh=_SC_MESH,
      scratch_types=[
          pltpu.VMEM((NCHUNK, CH), jnp.int32),
          pltpu.VMEM((NCHUNK, CH), jnp.int32),
          pltpu.VMEM((NCHUNK, CH), jnp.float32),
          pltpu.VMEM((CH + L,), jnp.float32),
          pltpu.VMEM((CH, AW), jnp.float32),
          pltpu.VMEM((CH, AW), jnp.float32),
          pltpu.VMEM((CH, AW), jnp.float32),
          pltpu.VMEM((CH, AW), jnp.float32),
          pltpu.VMEM((NZB, AW), jnp.float32),
          pltpu.VMEM_SHARED((N, AW), jnp.float32),
          pltpu.SemaphoreType.DMA,
          pltpu.SemaphoreType.DMA,
          pltpu.SemaphoreType.DMA,
          pltpu.SemaphoreType.DMA,
          pltpu.SemaphoreType.DMA,
      ],
      compiler_params=_SC_PARAMS,
  )
  return k(xa, src2, dst2, ae2)


def _mid_body(acc_ref, w1_ref, b1_ref, w2_ref, b2_ref, w2p_ref, hw1_ref):
  s = acc_ref[0] + acc_ref[1]
  h = s[:, :H1] / (s[:, H1:H1 + 1] + 1e-16)
  h = jnp.maximum(h, 0.0)
  w2p_ref[...] = jnp.dot(h, w2_ref[...],
                         preferred_element_type=jnp.float32) + b2_ref[...]
  hw1_ref[...] = jnp.dot(h, w1_ref[...],
                         preferred_element_type=jnp.float32) + b1_ref[...]


def _mid(acc, w1, b12, w2, b22):
  rb = 1000
  return pl.pallas_call(
      _mid_body,
      grid=(N // rb,),
      in_specs=[
          pl.BlockSpec((NC, rb, AW), lambda i: (0, i, 0)),
          pl.BlockSpec((H1, H2), lambda i: (0, 0)),
          pl.BlockSpec((1, H2), lambda i: (0, 0)),
          pl.BlockSpec((H1, H2), lambda i: (0, 0)),
          pl.BlockSpec((1, H2), lambda i: (0, 0)),
      ],
      out_specs=[
          pl.BlockSpec((rb, H2), lambda i: (i, 0)),
          pl.BlockSpec((rb, H2), lambda i: (i, 0)),
      ],
      out_shape=[
          jax.ShapeDtypeStruct((N, H2), jnp.float32),
          jax.ShapeDtypeStruct((N, H2), jnp.float32),
      ],
  )(acc, w1, b12, w2, b22)


def _aggr_sc_body(w2p_hbm, src2_hbm, dst2_hbm, gate_hbm, out_hbm,
                  srcb_v, dstb_v, rows_a, rows_b, gate_a, gate_b,
                  msg_a, msg_b, zb_v, acc_sh, lsem, ga, gb, ha, hb, sa, sb):
  cid = lax.axis_index("c")
  sid = lax.axis_index("s")
  wid = cid * NS + sid
  row0 = wid * NCHUNK
  ebase = wid * EW

  d1 = pltpu.async_copy(src2_hbm.at[pl.ds(row0, NCHUNK)], srcb_v, lsem)
  d2 = pltpu.async_copy(dst2_hbm.at[pl.ds(row0, NCHUNK)], dstb_v, lsem)

  @pl.loop(0, NZB)
  def _(i):
    for c in range(H2 // L):
      zb_v[i, pl.ds(c * L, L)] = jnp.zeros((L,), jnp.float32)

  for m in range(NPT_A // NZB):
    pltpu.sync_copy(zb_v, acc_sh.at[pl.ds(sid * NPT_A + m * NZB, NZB)])

  @pl.when(sid == NS - 1)
  def _():
    pltpu.sync_copy(zb_v.at[pl.ds(0, NPT_B - NPT_A)],
                    acc_sh.at[pl.ds(N - (NPT_B - NPT_A), NPT_B - NPT_A)])

  d1.wait()
  d2.wait()
  plsc.subcore_barrier()

  def compute(rows, gate, msg):
    @pl.loop(0, CH, unroll=8)
    def _(j):
      for c in range(H2 // L):
        sl = pl.ds(c * L, L)
        msg[j, sl] = rows[j, sl] * gate[j, sl]

  pltpu.async_copy(w2p_hbm.at[srcb_v.at[0]], rows_a, ga)
  pltpu.async_copy(gate_hbm.at[pl.ds(ebase, CH)], gate_a, ha)

  @pl.loop(0, NPAIR)
  def _(k):
    c0 = 2 * k
    pltpu.async_copy(w2p_hbm.at[srcb_v.at[c0 + 1]], rows_b, gb)
    pltpu.async_copy(gate_hbm.at[pl.ds(ebase + (c0 + 1) * CH, CH)], gate_b, hb)
    pltpu.make_async_copy(w2p_hbm.at[srcb_v.at[c0]], rows_a, ga).wait()
    pltpu.make_async_copy(gate_hbm.at[pl.ds(ebase, CH)], gate_a, ha).wait()

    @pl.when(k > 0)
    def _():
      pltpu.make_async_copy(msg_a, acc_sh.at[dstb_v.at[c0 - 2]], sa).wait()

    compute(rows_a, gate_a, msg_a)
    pltpu.async_copy(msg_a, acc_sh.at[dstb_v.at[c0]], sa, add=True)
    pltpu.async_copy(w2p_hbm.at[srcb_v.at[c0 + 2]], rows_a, ga)
    pltpu.async_copy(gate_hbm.at[pl.ds(ebase + (c0 + 2) * CH, CH)], gate_a, ha)
    pltpu.make_async_copy(w2p_hbm.at[srcb_v.at[c0 + 1]], rows_b, gb).wait()
    pltpu.make_async_copy(gate_hbm.at[pl.ds(ebase, CH)], gate_b, hb).wait()

    @pl.when(k > 0)
    def _():
      pltpu.make_async_copy(msg_b, acc_sh.at[dstb_v.at[c0 - 1]], sb).wait()

    compute(rows_b, gate_b, msg_b)
    pltpu.async_copy(msg_b, acc_sh.at[dstb_v.at[c0 + 1]], sb, add=True)

  last = NCHUNK - 1
  pltpu.make_async_copy(w2p_hbm.at[srcb_v.at[last]], rows_a, ga).wait()
  pltpu.make_async_copy(gate_hbm.at[pl.ds(ebase, CH)], gate_a, ha).wait()
  pltpu.make_async_copy(msg_a, acc_sh.at[dstb_v.at[last - 2]], sa).wait()
  compute(rows_a, gate_a, msg_a)
  pltpu.async_copy(msg_a, acc_sh.at[dstb_v.at[last]], sa, add=True)
  pltpu.make_async_copy(msg_a, acc_sh.at[dstb_v.at[last]], sa).wait()
  pltpu.make_async_copy(msg_b, acc_sh.at[dstb_v.at[last - 1]], sb).wait()

  plsc.subcore_barrier()

  for m in range(NPT_A // NZB):
    pltpu.sync_copy(acc_sh.at[pl.ds(sid * NPT_A + m * NZB, NZB)], zb_v)
    pltpu.sync_copy(zb_v, out_hbm.at[cid, pl.ds(sid * NPT_A + m * NZB, NZB)])

  @pl.when(sid == NS - 1)
  def _():
    pltpu.sync_copy(acc_sh.at[pl.ds(N - (NPT_B - NPT_A), NPT_B - NPT_A)],
                    zb_v.at[pl.ds(0, NPT_B - NPT_A)])
    pltpu.sync_copy(zb_v.at[pl.ds(0, NPT_B - NPT_A)],
                    out_hbm.at[cid, pl.ds(N - (NPT_B - NPT_A), NPT_B - NPT_A)])


def _aggr_sc(w2p, src2, dst2, gate):
  k = pl.kernel(
      _aggr_sc_body,
      out_type=jax.ShapeDtypeStruct((NC, N, H2), jnp.float32),
      mesh=_SC_MESH,
      scratch_types=[
          pltpu.VMEM((NCHUNK, CH), jnp.int32),
          pltpu.VMEM((NCHUNK, CH), jnp.int32),
          pltpu.VMEM((CH, H2), jnp.float32),
          pltpu.VMEM((CH, H2), jnp.float32),
          pltpu.VMEM((CH, 2 * H2), jnp.float32),
          pltpu.VMEM((CH, 2 * H2), jnp.float32),
          pltpu.VMEM((CH, H2), jnp.float32),
          pltpu.VMEM((CH, H2), jnp.float32),
          pltpu.VMEM((NZB, H2), jnp.float32),
          pltpu.VMEM_SHARED((N, H2), jnp.float32),
          pltpu.SemaphoreType.DMA,
          pltpu.SemaphoreType.DMA,
          pltpu.SemaphoreType.DMA,
          pltpu.SemaphoreType.DMA,
          pltpu.SemaphoreType.DMA,
          pltpu.SemaphoreType.DMA,
          pltpu.SemaphoreType.DMA,
      ],
      compiler_params=_SC_PARAMS,
  )
  return k(w2p, src2, dst2, gate)


def _head_body(hw1_ref, aggr_ref, g_ref, b_ref, out_ref):
  hh = hw1_ref[...] + aggr_ref[0] + aggr_ref[1]
  mean = jnp.mean(hh, axis=-1, keepdims=True)
  d = hh - mean
  var = jnp.mean(d * d, axis=-1, keepdims=True)
  y = d * lax.rsqrt(var + 1e-5) * g_ref[...] + b_ref[...]
  y = jnp.maximum(y, 0.0)
  m = jnp.max(y, axis=-1, keepdims=True)
  ey = jnp.exp(y - m)
  sm = ey / jnp.sum(ey, axis=-1, keepdims=True)
  o = jnp.sqrt(sm + 1e-8)
  nrm = jnp.sqrt(jnp.sum(o * o, axis=-1, keepdims=True))
  out_ref[...] = o / nrm


def _head(hw1, aggr, g2, b2):
  rb = 1000
  return pl.pallas_call(
      _head_body,
      grid=(N // rb,),
      in_specs=[
          pl.BlockSpec((rb, H2), lambda i: (i, 0)),
          pl.BlockSpec((NC, rb, H2), lambda i: (0, i, 0)),
          pl.BlockSpec((1, H2), lambda i: (0, 0)),
          pl.BlockSpec((1, H2), lambda i: (0, 0)),
      ],
      out_specs=pl.BlockSpec((rb, H2), lambda i: (i, 0)),
      out_shape=jax.ShapeDtypeStruct((N, H2), jnp.float32),
  )(hw1, aggr, g2, b2)


def kernel(x, edge_index, edge_attr, Wn, bn, We, be, attn, W1, b1, W2, b2,
           cw1, cb1, cw2, cb2, fcW, fcb, gamma, beta):
  av = attn[0, 0]
  wa = We @ av
  ba = be @ av
  m1 = jnp.zeros((16, H1), jnp.float32)
  m2 = jnp.zeros((H1, 8), jnp.float32)
  for t in range(3):
    s_t = jnp.eye(8, 8, 1 - t, dtype=jnp.float32)
    m1 = m1 + jnp.kron(cw1[:, :, t].T, s_t)
    m2 = m2 + jnp.kron(cw2[0, :, t:t + 1], s_t)
  b1c = jnp.repeat(cb1, 8)

  xa = _node_pre(x, Wn, bn.reshape(1, H1), av.reshape(H1, 1))
  ae2 = _ae_pre(edge_attr, wa.reshape(16, 1), ba.reshape(1, 1))
  fcw2 = jnp.concatenate([fcW, fcW], axis=1)
  fcb2w = jnp.concatenate([fcb, fcb]).reshape(1, 2 * H2)
  gate = _gate_pre(edge_attr, m1, b1c.reshape(1, H1), m2, cb2.reshape(1, 1),
                   fcw2, fcb2w)
  src2 = edge_index[0].reshape(E // CH, CH)
  dst2 = edge_index[1].reshape(E // CH, CH)
  acc = _gat_sc(xa, src2, dst2, ae2)
  w2p, hw1 = _mid(acc, W1, b1.reshape(1, H2), W2, b2.reshape(1, H2))
  aggr = _aggr_sc(w2p, src2, dst2, gate)
  return _head(hw1, aggr, gamma.reshape(1, H2), beta.reshape(1, H2))
